# baseline (speedup 1.0000x reference)
import functools
import math

import jax
import jax.numpy as jnp
from jax import lax
from jax.experimental import pallas as pl
from jax.experimental.pallas import tpu as pltpu
from jax.experimental.pallas import tpu_sc as plsc

S = 64.0
MARGIN = 0.5
COS_M = math.cos(MARGIN)
SIN_M = math.sin(MARGIN)
THETA = math.cos(math.pi - MARGIN)
SINMM = math.sin(math.pi - MARGIN) * MARGIN
C1 = S / math.log(2.0)

B_TC = 256
TC_BR = 32
SC_CW = 3456
SC_NBUF = 4
L16 = 16


def _tc_body(labels_ref, logits_ref, rowsum_ref, t_ref, *, V, BR):
    x = logits_ref[...]
    cols = lax.broadcasted_iota(jnp.int32, (BR, V), 1)
    lbl = labels_ref[...]
    safe_lbl = jnp.where(lbl < 0, 0, lbl)
    x = jnp.where(cols < V, x, -1.0)
    e = jnp.exp2(C1 * x - C1)
    rowsum_ref[...] = jnp.sum(e, axis=1, keepdims=True)
    t_ref[...] = jnp.sum(jnp.where(cols == safe_lbl, x, 0.0), axis=1, keepdims=True)


def _tc_pass(logits, labels_2d, Btc):
    B, V = logits.shape
    nsteps = Btc // TC_BR
    body = functools.partial(_tc_body, V=V, BR=TC_BR)
    return pl.pallas_call(
        body,
        grid=(nsteps,),
        in_specs=[
            pl.BlockSpec((TC_BR, 1), lambda i: (i, 0)),
            pl.BlockSpec((TC_BR, V), lambda i: (i, 0)),
        ],
        out_specs=[
            pl.BlockSpec((TC_BR, 1), lambda i: (i, 0)),
            pl.BlockSpec((TC_BR, 1), lambda i: (i, 0)),
        ],
        out_shape=[
            jax.ShapeDtypeStruct((Btc, 1), jnp.float32),
            jax.ShapeDtypeStruct((Btc, 1), jnp.float32),
        ],
    )(labels_2d, logits)


def _make_sc_dense(B, V, r_start, num_cores, num_subcores):
    nw = num_cores * num_subcores
    R_sc = B - r_start
    rows_per_tec = R_sc // nw
    slabs = rows_per_tec // 8
    assert rows_per_tec % 8 == 0 and R_sc % nw == 0
    nch = V // SC_CW
    assert nch % SC_NBUF == 0
    rem = V - nch * SC_CW
    mid_w = (rem // 128) * 128
    mid0 = nch * SC_CW
    tail0 = mid0 + mid_w
    tailw = V - tail0
    gpc = SC_CW // L16

    mesh = plsc.VectorSubcoreMesh(core_axis_name="c", subcore_axis_name="s")

    @functools.partial(
        pl.kernel,
        out_type=(
            jax.ShapeDtypeStruct((R_sc, L16), jnp.float32),
            jax.ShapeDtypeStruct((R_sc, L16), jnp.float32),
        ),
        mesh=mesh,
        scratch_types=[
            [pltpu.VMEM((8, SC_CW), jnp.float32) for _ in range(SC_NBUF)],
            pltpu.VMEM((8, tailw), jnp.float32),
            pltpu.VMEM((8, 8, 128), jnp.float32),
            pltpu.VMEM((8, L16), jnp.float32),
            pltpu.VMEM((8, L16), jnp.float32),
            pltpu.VMEM((((rows_per_tec + L16 - 1) // L16) * L16,), jnp.int32),
            [pltpu.SemaphoreType.DMA for _ in range(SC_NBUF)],
            pltpu.SemaphoreType.DMA,
        ],
    )
    def sc_dense(
        labels_hbm,
        logits_hbm,
        part_hbm,
        tpart_hbm,
        bufs,
        tail_v,
        ttile_v,
        acc_v,
        tacc_v,
        lbl_v,
        sems,
        sem_t,
    ):
        wid = lax.axis_index("s") * num_cores + lax.axis_index("c")
        my_lbl0 = wid * rows_per_tec
        pltpu.sync_copy(
            labels_hbm.at[pl.ds(r_start + my_lbl0, rows_per_tec)],
            lbl_v.at[pl.ds(0, rows_per_tec)],
        )
        lane16 = lax.iota(jnp.int32, L16)

        for sl in range(slabs):
            r0 = r_start + (wid * slabs + sl) * 8
            o0 = (wid * slabs + sl) * 8

            safe_cols = []
            for r in range(8):
                j = sl * 8 + r
                g16 = (j // L16) * L16
                lbl_r = lbl_v[pl.ds(g16, L16)][j % L16]
                lbl_r = jnp.where(lbl_r < 0, 0, lbl_r)
                safe_cols.append(lbl_r)
                cw0 = (lbl_r // 128) * 128
                pltpu.make_async_copy(
                    logits_hbm.at[pl.ds(r0, 8), pl.ds(cw0, 128)],
                    ttile_v.at[r],
                    sem_t,
                ).start()

            def chunk_src(c):
                return logits_hbm.at[pl.ds(r0, 8), pl.ds(c * SC_CW, SC_CW)]

            for b in range(SC_NBUF):
                pltpu.make_async_copy(chunk_src(b), bufs[b], sems[b]).start()

            def accumulate(buf, ngroups, accs):
                def inner(g, a):
                    out = []
                    for r in range(8):
                        v = buf[r, pl.ds(g * L16, L16)]
                        out.append(a[r] + jnp.exp(S * v - S))
                    return tuple(out)

                return lax.fori_loop(0, ngroups, inner, accs)

            def ring_body(k, accs):
                for b in range(SC_NBUF):
                    c = SC_NBUF * k + b
                    pltpu.make_async_copy(chunk_src(c), bufs[b], sems[b]).wait()
                    accs = accumulate(bufs[b], gpc, accs)

                    @pl.when(c + SC_NBUF < nch)
                    def _():
                        pltpu.make_async_copy(
                            chunk_src(c + SC_NBUF), bufs[b], sems[b]
                        ).start()

                return accs

            accs = tuple(jnp.zeros((L16,), jnp.float32) for _ in range(8))
            accs = lax.fori_loop(0, nch // SC_NBUF, ring_body, accs)

            if mid_w:
                mid_dst = bufs[0].at[pl.ds(0, 8), pl.ds(0, mid_w)]
                pltpu.sync_copy(
                    logits_hbm.at[pl.ds(r0, 8), pl.ds(mid0, mid_w)], mid_dst
                )
                accs = accumulate(bufs[0], mid_w // L16, accs)

            if tailw:
                pltpu.sync_copy(
                    logits_hbm.at[pl.ds(r0, 8), pl.ds(tail0, tailw)], tail_v
                )
                accs = list(accs)
                for g in range(tailw // L16):
                    for r in range(8):
                        v = tail_v[r, pl.ds(g * L16, L16)]
                        accs[r] = accs[r] + jnp.exp(S * v - S)

            for r in range(8):
                acc_v[r, :] = accs[r]

            for r in range(8):
                pltpu.make_async_copy(
                    logits_hbm.at[pl.ds(r0, 8), pl.ds(0, 128)],
                    ttile_v.at[r],
                    sem_t,
                ).wait()
            for r in range(8):
                lbl_r = safe_cols[r]
                lane = lbl_r % 128
                g = (lane // L16) * L16
                vec = ttile_v[r, r, pl.ds(g, L16)]
                tacc_v[r, :] = jnp.where(lane16 == lane % L16, vec, 0.0)

            pltpu.sync_copy(acc_v, part_hbm.at[pl.ds(o0, 8), :])
            pltpu.sync_copy(tacc_v, tpart_hbm.at[pl.ds(o0, 8), :])

    return sc_dense


def _loss_terms(rowsum, t, labels):
    sin_t = jnp.sqrt(jnp.maximum(1.0 - t * t, 0.0))
    new_t = jnp.where(t > THETA, t * COS_M - sin_t * SIN_M, t - SINMM)
    new_t = jnp.where(labels != -1, new_t, t)
    adj = rowsum - jnp.exp2(C1 * t - C1) + jnp.exp2(C1 * new_t - C1)
    adj = jnp.maximum(adj, 1e-35)
    lse = S + jnp.log(adj)
    return jnp.sum(lse - S * new_t)


def _combine_body(rs_tc_ref, t_tc_ref, part_ref, tpart_ref, labels_ref, out_ref, *, B):
    rs_sc = jnp.sum(part_ref[...], axis=1, keepdims=True)
    t_sc = jnp.sum(tpart_ref[...], axis=1, keepdims=True)
    btc = rs_tc_ref.shape[0]
    tot = _loss_terms(rs_tc_ref[...], t_tc_ref[...], labels_ref[0:btc, :])
    tot += _loss_terms(rs_sc, t_sc, labels_ref[btc:, :])
    out_ref[0, 0] = tot * (1.0 / B)


def kernel(logits, labels):
    B, V = logits.shape
    labels_i32 = labels.astype(jnp.int32)
    labels_2d = labels_i32.reshape(B, 1)
    info = plsc.get_sparse_core_info()

    sc_dense = _make_sc_dense(B, V, B_TC, info.num_cores, info.num_subcores)
    part, tpart = sc_dense(labels_i32, logits)
    rs_tc, t_tc = _tc_pass(logits, labels_2d, B_TC)

    out = pl.pallas_call(
        functools.partial(_combine_body, B=B),
        out_specs=pl.BlockSpec(memory_space=pltpu.SMEM),
        out_shape=jax.ShapeDtypeStruct((1, 1), jnp.float32),
    )(rs_tc, t_tc, part, tpart, labels_2d)
    return out[0, 0]


# speedup vs baseline: 1.1300x; 1.1300x over previous
import functools
import math

import jax
import jax.numpy as jnp
from jax import lax
from jax.experimental import pallas as pl
from jax.experimental.pallas import tpu as pltpu
from jax.experimental.pallas import tpu_sc as plsc

S = 64.0
MARGIN = 0.5
COS_M = math.cos(MARGIN)
SIN_M = math.sin(MARGIN)
THETA = math.cos(math.pi - MARGIN)
SINMM = math.sin(math.pi - MARGIN) * MARGIN
C1 = S / math.log(2.0)

B_TC = 512
TC_BR = 32
SC_CW = 3456
SC_NBUF = 4
L16 = 16


def _tc_body(labels_ref, logits_ref, rowsum_ref, t_ref, *, V, BR):
    x = logits_ref[...]
    cols = lax.broadcasted_iota(jnp.int32, (BR, V), 1)
    lbl = labels_ref[...]
    safe_lbl = jnp.where(lbl < 0, 0, lbl)
    x = jnp.where(cols < V, x, -1.0)
    e = jnp.exp2(C1 * x - C1)
    rowsum_ref[...] = jnp.sum(e, axis=1, keepdims=True)
    t_ref[...] = jnp.sum(jnp.where(cols == safe_lbl, x, 0.0), axis=1, keepdims=True)


def _tc_pass(logits, labels_2d, Btc):
    B, V = logits.shape
    nsteps = Btc // TC_BR
    body = functools.partial(_tc_body, V=V, BR=TC_BR)
    return pl.pallas_call(
        body,
        grid=(nsteps,),
        in_specs=[
            pl.BlockSpec((TC_BR, 1), lambda i: (i, 0)),
            pl.BlockSpec((TC_BR, V), lambda i: (i, 0)),
        ],
        out_specs=[
            pl.BlockSpec((TC_BR, 1), lambda i: (i, 0)),
            pl.BlockSpec((TC_BR, 1), lambda i: (i, 0)),
        ],
        out_shape=[
            jax.ShapeDtypeStruct((Btc, 1), jnp.float32),
            jax.ShapeDtypeStruct((Btc, 1), jnp.float32),
        ],
    )(labels_2d, logits)


def _make_sc_dense(B, V, r_start, num_cores, num_subcores):
    nw = num_cores * num_subcores
    R_sc = B - r_start
    rows_per_tec = R_sc // nw
    slabs = rows_per_tec // 8
    assert rows_per_tec % 8 == 0 and R_sc % nw == 0
    nch = V // SC_CW
    assert nch % SC_NBUF == 0
    rem = V - nch * SC_CW
    mid_w = (rem // 128) * 128
    mid0 = nch * SC_CW
    tail0 = mid0 + mid_w
    tailw = V - tail0
    gpc = SC_CW // L16

    mesh = plsc.VectorSubcoreMesh(core_axis_name="c", subcore_axis_name="s")

    @functools.partial(
        pl.kernel,
        out_type=(
            jax.ShapeDtypeStruct((R_sc, L16), jnp.float32),
            jax.ShapeDtypeStruct((R_sc, L16), jnp.float32),
        ),
        mesh=mesh,
        scratch_types=[
            [pltpu.VMEM((8, SC_CW), jnp.float32) for _ in range(SC_NBUF)],
            pltpu.VMEM((8, tailw), jnp.float32),
            pltpu.VMEM((8, 8, 128), jnp.float32),
            pltpu.VMEM((8, L16), jnp.float32),
            pltpu.VMEM((8, L16), jnp.float32),
            pltpu.VMEM((((rows_per_tec + L16 - 1) // L16) * L16,), jnp.int32),
            [pltpu.SemaphoreType.DMA for _ in range(SC_NBUF)],
            pltpu.SemaphoreType.DMA,
        ],
    )
    def sc_dense(
        labels_hbm,
        logits_hbm,
        part_hbm,
        tpart_hbm,
        bufs,
        tail_v,
        ttile_v,
        acc_v,
        tacc_v,
        lbl_v,
        sems,
        sem_t,
    ):
        wid = lax.axis_index("s") * num_cores + lax.axis_index("c")
        my_lbl0 = wid * rows_per_tec
        pltpu.sync_copy(
            labels_hbm.at[pl.ds(r_start + my_lbl0, rows_per_tec)],
            lbl_v.at[pl.ds(0, rows_per_tec)],
        )
        lane16 = lax.iota(jnp.int32, L16)

        for sl in range(slabs):
            r0 = r_start + (wid * slabs + sl) * 8
            o0 = (wid * slabs + sl) * 8

            safe_cols = []
            for r in range(8):
                j = sl * 8 + r
                g16 = (j // L16) * L16
                lbl_r = lbl_v[pl.ds(g16, L16)][j % L16]
                lbl_r = jnp.where(lbl_r < 0, 0, lbl_r)
                safe_cols.append(lbl_r)
                cw0 = (lbl_r // 128) * 128
                pltpu.make_async_copy(
                    logits_hbm.at[pl.ds(r0, 8), pl.ds(cw0, 128)],
                    ttile_v.at[r],
                    sem_t,
                ).start()

            def chunk_src(c):
                return logits_hbm.at[pl.ds(r0, 8), pl.ds(c * SC_CW, SC_CW)]

            for b in range(SC_NBUF):
                pltpu.make_async_copy(chunk_src(b), bufs[b], sems[b]).start()

            def accumulate(buf, ngroups, accs):
                def inner(g, a):
                    out = []
                    for r in range(8):
                        v = buf[r, pl.ds(g * L16, L16)]
                        out.append(a[r] + jnp.exp(S * v - S))
                    return tuple(out)

                return lax.fori_loop(0, ngroups, inner, accs)

            def ring_body(k, accs):
                for b in range(SC_NBUF):
                    c = SC_NBUF * k + b
                    pltpu.make_async_copy(chunk_src(c), bufs[b], sems[b]).wait()
                    accs = accumulate(bufs[b], gpc, accs)

                    @pl.when(c + SC_NBUF < nch)
                    def _():
                        pltpu.make_async_copy(
                            chunk_src(c + SC_NBUF), bufs[b], sems[b]
                        ).start()

                return accs

            accs = tuple(jnp.zeros((L16,), jnp.float32) for _ in range(8))
            accs = lax.fori_loop(0, nch // SC_NBUF, ring_body, accs)

            if mid_w:
                mid_dst = bufs[0].at[pl.ds(0, 8), pl.ds(0, mid_w)]
                pltpu.sync_copy(
                    logits_hbm.at[pl.ds(r0, 8), pl.ds(mid0, mid_w)], mid_dst
                )
                accs = accumulate(bufs[0], mid_w // L16, accs)

            if tailw:
                pltpu.sync_copy(
                    logits_hbm.at[pl.ds(r0, 8), pl.ds(tail0, tailw)], tail_v
                )
                accs = list(accs)
                for g in range(tailw // L16):
                    for r in range(8):
                        v = tail_v[r, pl.ds(g * L16, L16)]
                        accs[r] = accs[r] + jnp.exp(S * v - S)

            for r in range(8):
                acc_v[r, :] = accs[r]

            for r in range(8):
                pltpu.make_async_copy(
                    logits_hbm.at[pl.ds(r0, 8), pl.ds(0, 128)],
                    ttile_v.at[r],
                    sem_t,
                ).wait()
            for r in range(8):
                lbl_r = safe_cols[r]
                lane = lbl_r % 128
                g = (lane // L16) * L16
                vec = ttile_v[r, r, pl.ds(g, L16)]
                tacc_v[r, :] = jnp.where(lane16 == lane % L16, vec, 0.0)

            pltpu.sync_copy(acc_v, part_hbm.at[pl.ds(o0, 8), :])
            pltpu.sync_copy(tacc_v, tpart_hbm.at[pl.ds(o0, 8), :])

    return sc_dense


def _loss_terms(rowsum, t, labels):
    sin_t = jnp.sqrt(jnp.maximum(1.0 - t * t, 0.0))
    new_t = jnp.where(t > THETA, t * COS_M - sin_t * SIN_M, t - SINMM)
    new_t = jnp.where(labels != -1, new_t, t)
    adj = rowsum - jnp.exp2(C1 * t - C1) + jnp.exp2(C1 * new_t - C1)
    adj = jnp.maximum(adj, 1e-35)
    lse = S + jnp.log(adj)
    return jnp.sum(lse - S * new_t)


def _combine_body(rs_tc_ref, t_tc_ref, part_ref, tpart_ref, labels_ref, out_ref, *, B):
    rs_sc = jnp.sum(part_ref[...], axis=1, keepdims=True)
    t_sc = jnp.sum(tpart_ref[...], axis=1, keepdims=True)
    btc = rs_tc_ref.shape[0]
    tot = _loss_terms(rs_tc_ref[...], t_tc_ref[...], labels_ref[0:btc, :])
    tot += _loss_terms(rs_sc, t_sc, labels_ref[btc:, :])
    out_ref[0, 0] = tot * (1.0 / B)


def kernel(logits, labels):
    B, V = logits.shape
    labels_i32 = labels.astype(jnp.int32)
    labels_2d = labels_i32.reshape(B, 1)
    info = plsc.get_sparse_core_info()

    sc_dense = _make_sc_dense(B, V, B_TC, info.num_cores, info.num_subcores)
    part, tpart = sc_dense(labels_i32, logits)
    rs_tc, t_tc = _tc_pass(logits, labels_2d, B_TC)

    out = pl.pallas_call(
        functools.partial(_combine_body, B=B),
        out_specs=pl.BlockSpec(memory_space=pltpu.SMEM),
        out_shape=jax.ShapeDtypeStruct((1, 1), jnp.float32),
    )(rs_tc, t_tc, part, tpart, labels_2d)
    return out[0, 0]
